# Initial kernel scaffold; baseline (speedup 1.0000x reference)
import jax
import jax.numpy as jnp
from jax.experimental import pallas as pl


def kernel(features_1, edge_index_1, batch_1, features_2, edge_index_2, batch_2, params):
    raise NotImplementedError("write your pallas kernel here")


# speedup vs baseline: 3.5249x; 3.5249x over previous
import functools

import jax
import jax.numpy as jnp
from jax import lax
from jax.experimental import pallas as pl
from jax.experimental.pallas import tpu as pltpu
from jax.experimental.pallas import tpu_sc as plsc

N = 10000
E = 320000
B = 128
NS = 16
TPE = E // NS
CHUNK = 80
NCH = TPE // CHUNK
RPT = 640
WB = 80


@functools.cache
def _make_agg(d: int):
    mesh = plsc.VectorSubcoreMesh(core_axis_name="c", subcore_axis_name="s")

    @functools.partial(
        pl.kernel,
        out_type=[jax.ShapeDtypeStruct((N, d), jnp.float32),
                  jax.ShapeDtypeStruct((N, d), jnp.float32)],
        mesh=mesh,
        scratch_types=[
            pltpu.VMEM_SHARED((N, d), jnp.float32),
            pltpu.VMEM((2, CHUNK), jnp.int32),
            pltpu.VMEM((2, CHUNK), jnp.int32),
            pltpu.VMEM((2, CHUNK, d), jnp.float32),
            pltpu.VMEM((WB, d), jnp.float32),
            pltpu.SemaphoreType.DMA((2,)),
            pltpu.SemaphoreType.DMA((2,)),
        ],
        compiler_params=pltpu.CompilerParams(use_tc_tiling_on_sc=False),
    )
    def agg_kernel(x1, s1, d1, x2, s2, d2, o1, o2,
                   acc, srcv, dstv, rows, zbuf, gsem, ssem):
        cid = lax.axis_index("c")
        sid = lax.axis_index("s")

        def zrow(r, carry):
            for cc in range(d // 16):
                zbuf[r, pl.ds(cc * 16, 16)] = jnp.zeros((16,), jnp.float32)
            return carry

        lax.fori_loop(0, WB, zrow, 0)
        base = sid * RPT
        nrep = jnp.where(sid == NS - 1, (N - (NS - 1) * RPT) // WB, RPT // WB)

        def zcopy(k, carry):
            off = pl.multiple_of(base + k * WB, WB)
            pltpu.sync_copy(zbuf, acc.at[pl.ds(off, WB), :])
            return carry

        lax.fori_loop(0, nrep, zcopy, 0)
        plsc.subcore_barrier()

        def work(x_ref, src_ref, dst_ref, out_ref):
            e0 = sid * TPE

            def body(i, carry):
                for b in range(2):
                    j = i * 2 + b

                    @pl.when(j >= 2)
                    def _wait_prev():
                        pltpu.make_async_copy(
                            rows.at[b], acc.at[dstv.at[b]], ssem.at[b]).wait()

                    off = pl.multiple_of(e0 + j * CHUNK, CHUNK)
                    pltpu.sync_copy(src_ref.at[pl.ds(off, CHUNK)], srcv.at[b])
                    pltpu.sync_copy(dst_ref.at[pl.ds(off, CHUNK)], dstv.at[b])
                    pltpu.async_copy(
                        x_ref.at[srcv.at[b]], rows.at[b], gsem.at[b]).wait()
                    pltpu.async_copy(
                        rows.at[b], acc.at[dstv.at[b]], ssem.at[b], add=True)
                return carry

            lax.fori_loop(0, NCH // 2, body, 0)
            for b in range(2):
                pltpu.make_async_copy(
                    rows.at[b], acc.at[dstv.at[b]], ssem.at[b]).wait()
            plsc.subcore_barrier()

            def wcopy(k, carry):
                off = pl.multiple_of(base + k * WB, WB)
                pltpu.sync_copy(acc.at[pl.ds(off, WB), :],
                                out_ref.at[pl.ds(off, WB), :])
                return carry

            lax.fori_loop(0, nrep, wcopy, 0)

        @pl.when(cid == 0)
        def _g1():
            work(x1, s1, d1, o1)

        @pl.when(cid == 1)
        def _g2():
            work(x2, s2, d2, o2)

    return agg_kernel


@functools.cache
def _make_layer(din: int, f: int):
    def body(x1, a1, x2, a2, bt1, bt2, eps, w1, b1, w2, b2, g, be,
             wi, bi, wo, bo, c1o, c2o, o1o, o2o):
        def graph(x_ref, a_ref, bt_ref, c_ref, o_ref):
            x = x_ref[...]
            h = (1.0 + eps[0, 0]) * x + a_ref[...]
            h = jnp.maximum(
                jnp.dot(h, w1[...], preferred_element_type=jnp.float32)
                + b1[0, :], 0.0)
            h = jnp.dot(h, w2[...], preferred_element_type=jnp.float32) + b2[0, :]
            c = jnp.maximum(h * g[0, :] + be[0, :], 0.0)
            c_ref[...] = c
            inn = jnp.maximum(
                jnp.dot(c, wi[...], preferred_element_type=jnp.float32)
                + bi[0, :], 0.0)
            bt = bt_ref[0, :]
            oh = (lax.broadcasted_iota(jnp.int32, (B, N), 0)
                  == bt[None, :]).astype(jnp.float32)
            pool = jnp.dot(oh, inn, preferred_element_type=jnp.float32)
            o_ref[...] = jnp.maximum(
                jnp.dot(pool, wo[...], preferred_element_type=jnp.float32)
                + bo[0, :], 0.0)

        graph(x1, a1, bt1, c1o, o1o)
        graph(x2, a2, bt2, c2o, o2o)

    return pl.pallas_call(
        body,
        out_shape=[jax.ShapeDtypeStruct((N, f), jnp.float32),
                   jax.ShapeDtypeStruct((N, f), jnp.float32),
                   jax.ShapeDtypeStruct((B, f), jnp.float32),
                   jax.ShapeDtypeStruct((B, f), jnp.float32)],
    )


@functools.cache
def _make_head():
    FL, TN = 128, 16

    def body(p0a, p0b, p1a, p1b, p2a, p2b, c1w, c1b, c2w, c2b,
             ntnw, wbt, nbias, s1w, s1b, s2w, s2b, m1w, m1b, m2w, m2b,
             al, bt, out):
        def sig(v):
            return 1.0 / (1.0 + jnp.exp(-v))

        d0 = jnp.exp(-jnp.square(p0a[...] - p0b[...]))
        d1 = jnp.exp(-jnp.square(p1a[...] - p1b[...]))
        d2 = jnp.exp(-jnp.square(p2a[...] - p2b[...]))
        diff = jnp.concatenate([d0, d1, d2], axis=1)
        h = jnp.maximum(
            jnp.dot(diff, c1w[...], preferred_element_type=jnp.float32)
            + c1b[0, :], 0.0)
        srep = jnp.tanh(
            jnp.dot(h, c2w[...], preferred_element_type=jnp.float32)
            + c2b[0, :])
        o1 = p2a[...]
        o2 = p2b[...]
        cols = []
        for t in range(TN):
            wt = ntnw[pl.ds(t * FL, FL), :]
            tmp = jnp.dot(o1, wt, preferred_element_type=jnp.float32)
            cols.append(jnp.sum(tmp * o2, axis=1, keepdims=True))
        scoring = jnp.concatenate(cols, axis=1)
        comb = jnp.concatenate([o1, o2], axis=1)
        blk = jnp.dot(comb, wbt[...], preferred_element_type=jnp.float32)
        sim_rep = jnp.maximum(scoring + blk + nbias[0, :], 0.0)
        sh = jnp.maximum(
            jnp.dot(sim_rep, m1w[...], preferred_element_type=jnp.float32)
            + m1b[0, :], 0.0)
        sim_score = sig(
            jnp.dot(sh, m2w[...], preferred_element_type=jnp.float32)
            + m2b[0, :])
        sc = jnp.maximum(
            jnp.dot(srep, s1w[...], preferred_element_type=jnp.float32)
            + s1b[0, :], 0.0)
        score = sig(
            jnp.dot(sc, s2w[...], preferred_element_type=jnp.float32)
            + s2b[0, :])
        out[...] = al[0, 0] * score + bt[0, 0] * sim_score

    return pl.pallas_call(
        body,
        out_shape=jax.ShapeDtypeStruct((B, 1), jnp.float32),
    )


def _agg_pair(c1, c2, s1, d1, s2, d2):
    return _make_agg(c1.shape[1])(c1, s1, d1, c2, s2, d2)


def kernel(features_1, edge_index_1, batch_1, features_2, edge_index_2,
           batch_2, params):
    p = params
    s1 = edge_index_1[0]
    dd1 = edge_index_1[1]
    s2 = edge_index_2[0]
    dd2 = edge_index_2[1]
    bt1 = batch_1.astype(jnp.int32).reshape(1, N)
    bt2 = batch_2.astype(jnp.int32).reshape(1, N)

    bn_scale = 1.0 / jnp.sqrt(jnp.float32(1.0 + 1e-5))
    c1, c2 = features_1, features_2
    outs = []
    dims = [128, 64, 64]
    filters = [64, 64, 128]
    for i in range(3):
        lp = p["gnn"][i]
        mi = p["mlp_inner"][i]
        mo = p["mlp_outer"][i]
        a1, a2 = _agg_pair(c1, c2, s1, dd1, s2, dd2)
        c1, c2, o1, o2 = _make_layer(dims[i], filters[i])(
            c1, a1, c2, a2, bt1, bt2,
            lp["eps"].reshape(1, 1),
            lp["lin1"]["W"], lp["lin1"]["b"].reshape(1, -1),
            lp["lin2"]["W"], lp["lin2"]["b"].reshape(1, -1),
            (lp["bn_gamma"] * bn_scale).reshape(1, -1),
            lp["bn_beta"].reshape(1, -1),
            mi["W"], mi["b"].reshape(1, -1),
            mo["W"], mo["b"].reshape(1, -1),
        )
        outs.append((o1, o2))

    FL, TN = 128, 16
    ntnw = p["ntn_W"].transpose(2, 0, 1).reshape(TN * FL, FL)
    res = _make_head()(
        outs[0][0], outs[0][1], outs[1][0], outs[1][1], outs[2][0], outs[2][1],
        p["conv1"]["W"], p["conv1"]["b"].reshape(1, -1),
        p["conv2"]["W"], p["conv2"]["b"].reshape(1, -1),
        ntnw,
        p["ntn_Wb"].T,
        p["ntn_bias"].reshape(1, TN),
        p["score1"]["W"], p["score1"]["b"].reshape(1, -1),
        p["score2"]["W"], p["score2"]["b"].reshape(1, -1),
        p["sim1"]["W"], p["sim1"]["b"].reshape(1, -1),
        p["sim2"]["W"], p["sim2"]["b"].reshape(1, -1),
        p["alpha"].reshape(1, 1), p["beta"].reshape(1, 1),
    )
    return res.reshape(-1)


# speedup vs baseline: 4.0680x; 1.1541x over previous
import functools

import jax
import jax.numpy as jnp
from jax import lax
from jax.experimental import pallas as pl
from jax.experimental.pallas import tpu as pltpu
from jax.experimental.pallas import tpu_sc as plsc

N = 10000
E = 320000
B = 128
NS = 16
NP = N + 8
CHUNK = 128
EC = 2560
EPAD = EC * CHUNK
TPC = EC // NS
IB = 20
NBLK = TPC // IB
ZR = 16
RPT = 640
WB = 80


@functools.cache
def _make_agg(d: int):
    nbuf = 2 if d == 128 else 4
    mesh = plsc.VectorSubcoreMesh(core_axis_name="c", subcore_axis_name="s")

    @functools.partial(
        pl.kernel,
        out_type=[jax.ShapeDtypeStruct((N, d), jnp.float32),
                  jax.ShapeDtypeStruct((N, d), jnp.float32)],
        mesh=mesh,
        scratch_types=[
            pltpu.VMEM_SHARED((NP, d), jnp.float32),
            pltpu.VMEM((2, IB, CHUNK), jnp.int32),
            pltpu.VMEM((2, IB, CHUNK), jnp.int32),
            pltpu.VMEM((nbuf, CHUNK, d), jnp.float32),
            pltpu.VMEM((ZR, d), jnp.float32),
            pltpu.SemaphoreType.DMA((nbuf,)),
            pltpu.SemaphoreType.DMA((nbuf,)),
            pltpu.SemaphoreType.DMA((2,)),
            pltpu.SemaphoreType.DMA((2,)),
        ],
        compiler_params=pltpu.CompilerParams(use_tc_tiling_on_sc=False),
    )
    def agg_kernel(x1, s1, d1, x2, s2, d2, o1, o2,
                   acc, srcb, dstb, rows, zbuf, gsem, ssem, isems, isemd):
        cid = lax.axis_index("c")
        sid = lax.axis_index("s")

        def zrow(r, carry):
            for cc in range(d // 16):
                zbuf[r, pl.ds(cc * 16, 16)] = jnp.zeros((16,), jnp.float32)
            return carry

        lax.fori_loop(0, ZR, zrow, 0)
        base = sid * RPT
        nz = jnp.where(sid == NS - 1, (N - (NS - 1) * RPT) // ZR, RPT // ZR)

        def zcopy(k, carry):
            off = pl.multiple_of(base + k * ZR, ZR)
            pltpu.sync_copy(zbuf, acc.at[pl.ds(off, ZR), :])
            return carry

        lax.fori_loop(0, nz, zcopy, 0)
        plsc.subcore_barrier()

        def work(x_ref, src_ref, dst_ref, out_ref):
            c0 = sid * TPC

            def load_idx(k):
                ib = k % 2
                off = pl.multiple_of(c0 + k * IB, IB)
                pltpu.async_copy(src_ref.at[pl.ds(off, IB), :],
                                 srcb.at[ib], isems.at[ib])
                pltpu.async_copy(dst_ref.at[pl.ds(off, IB), :],
                                 dstb.at[ib], isemd.at[ib])

            load_idx(0)
            for k in range(NBLK):
                ib = k % 2
                pltpu.make_async_copy(src_ref.at[pl.ds(c0, IB), :],
                                      srcb.at[ib], isems.at[ib]).wait()
                pltpu.make_async_copy(dst_ref.at[pl.ds(c0, IB), :],
                                      dstb.at[ib], isemd.at[ib]).wait()
                if k + 1 < NBLK:
                    load_idx(k + 1)
                if k > 0:
                    pltpu.make_async_copy(rows.at[0], acc.at[dstb.at[0, 0]],
                                          ssem.at[0]).wait()
                pltpu.async_copy(x_ref.at[srcb.at[ib, 0]], rows.at[0],
                                 gsem.at[0])

                def group(g, carry):
                    for b in range(nbuf):
                        jl = g * nbuf + b
                        nb = (b + 1) % nbuf
                        njl = jl + 1

                        @pl.when(njl < IB)
                        def _issue_next():
                            if k == 0:
                                @pl.when(njl >= nbuf)
                                def _wait_buf():
                                    pltpu.make_async_copy(
                                        rows.at[nb], acc.at[dstb.at[0, 0]],
                                        ssem.at[nb]).wait()
                            else:
                                pltpu.make_async_copy(
                                    rows.at[nb], acc.at[dstb.at[0, 0]],
                                    ssem.at[nb]).wait()
                            pltpu.async_copy(x_ref.at[srcb.at[ib, njl]],
                                             rows.at[nb], gsem.at[nb])

                        pltpu.make_async_copy(
                            x_ref.at[srcb.at[ib, 0]], rows.at[b],
                            gsem.at[b]).wait()
                        pltpu.async_copy(rows.at[b], acc.at[dstb.at[ib, jl]],
                                         ssem.at[b], add=True)
                    return carry

                lax.fori_loop(0, IB // nbuf, group, 0)
            for b in range(nbuf):
                pltpu.make_async_copy(rows.at[b], acc.at[dstb.at[0, 0]],
                                      ssem.at[b]).wait()
            plsc.subcore_barrier()

            nw = jnp.where(sid == NS - 1, (N - (NS - 1) * RPT) // WB,
                           RPT // WB)

            def wcopy(k, carry):
                off = pl.multiple_of(base + k * WB, WB)
                pltpu.sync_copy(acc.at[pl.ds(off, WB), :],
                                out_ref.at[pl.ds(off, WB), :])
                return carry

            lax.fori_loop(0, nw, wcopy, 0)

        @pl.when(cid == 0)
        def _g1():
            work(x1, s1, d1, o1)

        @pl.when(cid == 1)
        def _g2():
            work(x2, s2, d2, o2)

    return agg_kernel


@functools.cache
def _make_layer(din: int, f: int):
    def body(x1, a1, x2, a2, bt1, bt2, eps, w1, b1, w2, b2, g, be,
             wi, bi, wo, bo, c1o, c2o, o1o, o2o):
        def graph(x_ref, a_ref, bt_ref, c_ref, o_ref):
            x = x_ref[pl.ds(0, N), :]
            h = (1.0 + eps[0, 0]) * x + a_ref[...]
            h = jnp.maximum(
                jnp.dot(h, w1[...], preferred_element_type=jnp.float32)
                + b1[0, :], 0.0)
            h = jnp.dot(h, w2[...], preferred_element_type=jnp.float32) + b2[0, :]
            c = jnp.maximum(h * g[0, :] + be[0, :], 0.0)
            c_ref[pl.ds(0, N), :] = c
            c_ref[pl.ds(N, NP - N), :] = jnp.zeros((NP - N, f), jnp.float32)
            inn = jnp.maximum(
                jnp.dot(c, wi[...], preferred_element_type=jnp.float32)
                + bi[0, :], 0.0)
            bt = bt_ref[0, :]
            oh = (lax.broadcasted_iota(jnp.int32, (B, N), 0)
                  == bt[None, :]).astype(jnp.float32)
            pool = jnp.dot(oh, inn, preferred_element_type=jnp.float32)
            o_ref[...] = jnp.maximum(
                jnp.dot(pool, wo[...], preferred_element_type=jnp.float32)
                + bo[0, :], 0.0)

        graph(x1, a1, bt1, c1o, o1o)
        graph(x2, a2, bt2, c2o, o2o)

    return pl.pallas_call(
        body,
        out_shape=[jax.ShapeDtypeStruct((NP, f), jnp.float32),
                   jax.ShapeDtypeStruct((NP, f), jnp.float32),
                   jax.ShapeDtypeStruct((B, f), jnp.float32),
                   jax.ShapeDtypeStruct((B, f), jnp.float32)],
    )


@functools.cache
def _make_head():
    FL, TN = 128, 16

    def body(p0a, p0b, p1a, p1b, p2a, p2b, c1w, c1b, c2w, c2b,
             ntnw, wbt, nbias, s1w, s1b, s2w, s2b, m1w, m1b, m2w, m2b,
             al, bt, out):
        def sig(v):
            return 1.0 / (1.0 + jnp.exp(-v))

        d0 = jnp.exp(-jnp.square(p0a[...] - p0b[...]))
        d1 = jnp.exp(-jnp.square(p1a[...] - p1b[...]))
        d2 = jnp.exp(-jnp.square(p2a[...] - p2b[...]))
        diff = jnp.concatenate([d0, d1, d2], axis=1)
        h = jnp.maximum(
            jnp.dot(diff, c1w[...], preferred_element_type=jnp.float32)
            + c1b[0, :], 0.0)
        srep = jnp.tanh(
            jnp.dot(h, c2w[...], preferred_element_type=jnp.float32)
            + c2b[0, :])
        o1 = p2a[...]
        o2 = p2b[...]
        cols = []
        for t in range(TN):
            wt = ntnw[pl.ds(t * FL, FL), :]
            tmp = jnp.dot(o1, wt, preferred_element_type=jnp.float32)
            cols.append(jnp.sum(tmp * o2, axis=1, keepdims=True))
        scoring = jnp.concatenate(cols, axis=1)
        comb = jnp.concatenate([o1, o2], axis=1)
        blk = jnp.dot(comb, wbt[...], preferred_element_type=jnp.float32)
        sim_rep = jnp.maximum(scoring + blk + nbias[0, :], 0.0)
        sh = jnp.maximum(
            jnp.dot(sim_rep, m1w[...], preferred_element_type=jnp.float32)
            + m1b[0, :], 0.0)
        sim_score = sig(
            jnp.dot(sh, m2w[...], preferred_element_type=jnp.float32)
            + m2b[0, :])
        sc = jnp.maximum(
            jnp.dot(srep, s1w[...], preferred_element_type=jnp.float32)
            + s1b[0, :], 0.0)
        score = sig(
            jnp.dot(sc, s2w[...], preferred_element_type=jnp.float32)
            + s2b[0, :])
        out[...] = al[0, 0] * score + bt[0, 0] * sim_score

    return pl.pallas_call(
        body,
        out_shape=jax.ShapeDtypeStruct((B, 1), jnp.float32),
    )


def _agg_pair(c1, c2, s1, d1, s2, d2):
    return _make_agg(c1.shape[1])(c1, s1, d1, c2, s2, d2)


def kernel(features_1, edge_index_1, batch_1, features_2, edge_index_2,
           batch_2, params):
    p = params
    epad = jnp.full((EPAD - E,), N, jnp.int32)

    def prep(row):
        return jnp.concatenate([row.astype(jnp.int32), epad]).reshape(EC, CHUNK)

    s1 = prep(edge_index_1[0])
    dd1 = prep(edge_index_1[1])
    s2 = prep(edge_index_2[0])
    dd2 = prep(edge_index_2[1])
    bt1 = batch_1.astype(jnp.int32).reshape(1, N)
    bt2 = batch_2.astype(jnp.int32).reshape(1, N)

    bn_scale = 1.0 / jnp.sqrt(jnp.float32(1.0 + 1e-5))
    fpad = jnp.zeros((NP - N, 128), jnp.float32)
    c1 = jnp.concatenate([features_1, fpad])
    c2 = jnp.concatenate([features_2, fpad])
    outs = []
    dims = [128, 64, 64]
    filters = [64, 64, 128]
    for i in range(3):
        lp = p["gnn"][i]
        mi = p["mlp_inner"][i]
        mo = p["mlp_outer"][i]
        a1, a2 = _agg_pair(c1, c2, s1, dd1, s2, dd2)
        c1, c2, o1, o2 = _make_layer(dims[i], filters[i])(
            c1, a1, c2, a2, bt1, bt2,
            lp["eps"].reshape(1, 1),
            lp["lin1"]["W"], lp["lin1"]["b"].reshape(1, -1),
            lp["lin2"]["W"], lp["lin2"]["b"].reshape(1, -1),
            (lp["bn_gamma"] * bn_scale).reshape(1, -1),
            lp["bn_beta"].reshape(1, -1),
            mi["W"], mi["b"].reshape(1, -1),
            mo["W"], mo["b"].reshape(1, -1),
        )
        outs.append((o1, o2))

    FL, TN = 128, 16
    ntnw = p["ntn_W"].transpose(2, 0, 1).reshape(TN * FL, FL)
    res = _make_head()(
        outs[0][0], outs[0][1], outs[1][0], outs[1][1], outs[2][0], outs[2][1],
        p["conv1"]["W"], p["conv1"]["b"].reshape(1, -1),
        p["conv2"]["W"], p["conv2"]["b"].reshape(1, -1),
        ntnw,
        p["ntn_Wb"].T,
        p["ntn_bias"].reshape(1, TN),
        p["score1"]["W"], p["score1"]["b"].reshape(1, -1),
        p["score2"]["W"], p["score2"]["b"].reshape(1, -1),
        p["sim1"]["W"], p["sim1"]["b"].reshape(1, -1),
        p["sim2"]["W"], p["sim2"]["b"].reshape(1, -1),
        p["alpha"].reshape(1, 1), p["beta"].reshape(1, 1),
    )
    return res.reshape(-1)


# speedup vs baseline: 4.1283x; 1.0148x over previous
import functools

import jax
import jax.numpy as jnp
from jax import lax
from jax.experimental import pallas as pl
from jax.experimental.pallas import tpu as pltpu
from jax.experimental.pallas import tpu_sc as plsc

N = 10000
E = 320000
B = 128
NS = 16
NP = N + 8
CHUNK = 128
EC = 2560
EPAD = EC * CHUNK
TPC = EC // NS
RPT = 640
WB = 80


@functools.cache
def _make_agg(d: int):
    nbuf = 2 if d == 128 else 4
    ib_n = 32 if d == 128 else 40
    nblk = TPC // ib_n
    mesh = plsc.VectorSubcoreMesh(core_axis_name="c", subcore_axis_name="s")

    @functools.partial(
        pl.kernel,
        out_type=[jax.ShapeDtypeStruct((N, d), jnp.float32),
                  jax.ShapeDtypeStruct((N, d), jnp.float32)],
        mesh=mesh,
        scratch_types=[
            pltpu.VMEM_SHARED((NP, d), jnp.float32),
            pltpu.VMEM((2, ib_n, CHUNK), jnp.int32),
            pltpu.VMEM((2, ib_n, CHUNK), jnp.int32),
            pltpu.VMEM((nbuf, CHUNK, d), jnp.float32),
            pltpu.SemaphoreType.DMA((nbuf,)),
            pltpu.SemaphoreType.DMA((nbuf,)),
            pltpu.SemaphoreType.DMA((2,)),
            pltpu.SemaphoreType.DMA((2,)),
        ],
        compiler_params=pltpu.CompilerParams(use_tc_tiling_on_sc=False),
    )
    def agg_kernel(x1, s1, d1, x2, s2, d2, o1, o2,
                   acc, srcb, dstb, rows, gsem, ssem, isems, isemd):
        cid = lax.axis_index("c")
        sid = lax.axis_index("s")
        base = sid * RPT
        nwb = jnp.where(sid == NS - 1, (N - (NS - 1) * RPT) // WB, RPT // WB)
        zsrc = rows.at[0, pl.ds(0, WB), :]

        def zrow(r, carry):
            for cc in range(d // 16):
                rows[0, r, pl.ds(cc * 16, 16)] = jnp.zeros((16,), jnp.float32)
            return carry

        lax.fori_loop(0, WB, zrow, 0)

        def zcopy(k, carry):
            off = pl.multiple_of(base + k * WB, WB)
            pltpu.async_copy(zsrc, acc.at[pl.ds(off, WB), :], gsem.at[0])
            return carry

        lax.fori_loop(0, nwb, zcopy, 0)

        def zdrain(k, carry):
            pltpu.make_async_copy(zsrc, acc.at[pl.ds(base, WB), :],
                                  gsem.at[0]).wait()
            return carry

        lax.fori_loop(0, nwb, zdrain, 0)
        plsc.subcore_barrier()

        def work(x_ref, src_ref, dst_ref, out_ref):
            c0 = sid * TPC

            def load_idx(k):
                ib = k % 2
                off = pl.multiple_of(c0 + k * ib_n, 8)
                pltpu.async_copy(src_ref.at[pl.ds(off, ib_n), :],
                                 srcb.at[ib], isems.at[ib])
                pltpu.async_copy(dst_ref.at[pl.ds(off, ib_n), :],
                                 dstb.at[ib], isemd.at[ib])

            load_idx(0)
            for k in range(nblk):
                ib = k % 2
                pltpu.make_async_copy(src_ref.at[pl.ds(c0, ib_n), :],
                                      srcb.at[ib], isems.at[ib]).wait()
                pltpu.make_async_copy(dst_ref.at[pl.ds(c0, ib_n), :],
                                      dstb.at[ib], isemd.at[ib]).wait()
                if k + 1 < nblk:
                    load_idx(k + 1)
                if k > 0:
                    pltpu.make_async_copy(rows.at[0], acc.at[dstb.at[0, 0]],
                                          ssem.at[0]).wait()
                pltpu.async_copy(x_ref.at[srcb.at[ib, 0]], rows.at[0],
                                 gsem.at[0])

                def group(g, carry):
                    for b in range(nbuf):
                        jl = g * nbuf + b
                        nb = (b + 1) % nbuf
                        njl = jl + 1

                        @pl.when(njl < ib_n)
                        def _issue_next():
                            if k == 0:
                                @pl.when(njl >= nbuf)
                                def _wait_buf():
                                    pltpu.make_async_copy(
                                        rows.at[nb], acc.at[dstb.at[0, 0]],
                                        ssem.at[nb]).wait()
                            else:
                                pltpu.make_async_copy(
                                    rows.at[nb], acc.at[dstb.at[0, 0]],
                                    ssem.at[nb]).wait()
                            pltpu.async_copy(x_ref.at[srcb.at[ib, njl]],
                                             rows.at[nb], gsem.at[nb])

                        pltpu.make_async_copy(
                            x_ref.at[srcb.at[ib, 0]], rows.at[b],
                            gsem.at[b]).wait()
                        pltpu.async_copy(rows.at[b], acc.at[dstb.at[ib, jl]],
                                         ssem.at[b], add=True)
                    return carry

                lax.fori_loop(0, ib_n // nbuf, group, 0)
            for b in range(nbuf):
                pltpu.make_async_copy(rows.at[b], acc.at[dstb.at[0, 0]],
                                      ssem.at[b]).wait()
            plsc.subcore_barrier()

            def wcopy(k, carry):
                off = pl.multiple_of(base + k * WB, WB)
                pltpu.async_copy(acc.at[pl.ds(off, WB), :],
                                 out_ref.at[pl.ds(off, WB), :], gsem.at[0])
                return carry

            lax.fori_loop(0, nwb, wcopy, 0)

            def wdrain(k, carry):
                pltpu.make_async_copy(acc.at[pl.ds(base, WB), :],
                                      out_ref.at[pl.ds(base, WB), :],
                                      gsem.at[0]).wait()
                return carry

            lax.fori_loop(0, nwb, wdrain, 0)

        @pl.when(cid == 0)
        def _g1():
            work(x1, s1, d1, o1)

        @pl.when(cid == 1)
        def _g2():
            work(x2, s2, d2, o2)

    return agg_kernel


@functools.cache
def _make_layer(din: int, f: int):
    def body(x1, a1, x2, a2, bt1, bt2, eps, w1, b1, w2, b2, g, be,
             wi, bi, wo, bo, c1o, c2o, o1o, o2o):
        def graph(x_ref, a_ref, bt_ref, c_ref, o_ref):
            x = x_ref[pl.ds(0, N), :]
            h = (1.0 + eps[0, 0]) * x + a_ref[...]
            h = jnp.maximum(
                jnp.dot(h, w1[...], preferred_element_type=jnp.float32)
                + b1[0, :], 0.0)
            h = jnp.dot(h, w2[...], preferred_element_type=jnp.float32) + b2[0, :]
            c = jnp.maximum(h * g[0, :] + be[0, :], 0.0)
            c_ref[pl.ds(0, N), :] = c
            c_ref[pl.ds(N, NP - N), :] = jnp.zeros((NP - N, f), jnp.float32)
            inn = jnp.maximum(
                jnp.dot(c, wi[...], preferred_element_type=jnp.float32)
                + bi[0, :], 0.0)
            bt = bt_ref[0, :]
            oh = (lax.broadcasted_iota(jnp.int32, (B, N), 0)
                  == bt[None, :]).astype(jnp.float32)
            pool = jnp.dot(oh, inn, preferred_element_type=jnp.float32)
            o_ref[...] = jnp.maximum(
                jnp.dot(pool, wo[...], preferred_element_type=jnp.float32)
                + bo[0, :], 0.0)

        graph(x1, a1, bt1, c1o, o1o)
        graph(x2, a2, bt2, c2o, o2o)

    return pl.pallas_call(
        body,
        out_shape=[jax.ShapeDtypeStruct((NP, f), jnp.float32),
                   jax.ShapeDtypeStruct((NP, f), jnp.float32),
                   jax.ShapeDtypeStruct((B, f), jnp.float32),
                   jax.ShapeDtypeStruct((B, f), jnp.float32)],
    )


@functools.cache
def _make_head():
    FL, TN = 128, 16

    def body(p0a, p0b, p1a, p1b, p2a, p2b, c1w, c1b, c2w, c2b,
             ntnw, wbt, nbias, s1w, s1b, s2w, s2b, m1w, m1b, m2w, m2b,
             al, bt, out):
        def sig(v):
            return 1.0 / (1.0 + jnp.exp(-v))

        d0 = jnp.exp(-jnp.square(p0a[...] - p0b[...]))
        d1 = jnp.exp(-jnp.square(p1a[...] - p1b[...]))
        d2 = jnp.exp(-jnp.square(p2a[...] - p2b[...]))
        diff = jnp.concatenate([d0, d1, d2], axis=1)
        h = jnp.maximum(
            jnp.dot(diff, c1w[...], preferred_element_type=jnp.float32)
            + c1b[0, :], 0.0)
        srep = jnp.tanh(
            jnp.dot(h, c2w[...], preferred_element_type=jnp.float32)
            + c2b[0, :])
        o1 = p2a[...]
        o2 = p2b[...]
        cols = []
        for t in range(TN):
            wt = ntnw[pl.ds(t * FL, FL), :]
            tmp = jnp.dot(o1, wt, preferred_element_type=jnp.float32)
            cols.append(jnp.sum(tmp * o2, axis=1, keepdims=True))
        scoring = jnp.concatenate(cols, axis=1)
        comb = jnp.concatenate([o1, o2], axis=1)
        blk = jnp.dot(comb, wbt[...], preferred_element_type=jnp.float32)
        sim_rep = jnp.maximum(scoring + blk + nbias[0, :], 0.0)
        sh = jnp.maximum(
            jnp.dot(sim_rep, m1w[...], preferred_element_type=jnp.float32)
            + m1b[0, :], 0.0)
        sim_score = sig(
            jnp.dot(sh, m2w[...], preferred_element_type=jnp.float32)
            + m2b[0, :])
        sc = jnp.maximum(
            jnp.dot(srep, s1w[...], preferred_element_type=jnp.float32)
            + s1b[0, :], 0.0)
        score = sig(
            jnp.dot(sc, s2w[...], preferred_element_type=jnp.float32)
            + s2b[0, :])
        out[...] = al[0, 0] * score + bt[0, 0] * sim_score

    return pl.pallas_call(
        body,
        out_shape=jax.ShapeDtypeStruct((B, 1), jnp.float32),
    )


def _agg_pair(c1, c2, s1, d1, s2, d2):
    return _make_agg(c1.shape[1])(c1, s1, d1, c2, s2, d2)


def kernel(features_1, edge_index_1, batch_1, features_2, edge_index_2,
           batch_2, params):
    p = params
    epad = jnp.full((EPAD - E,), N, jnp.int32)

    def prep(row):
        return jnp.concatenate([row.astype(jnp.int32), epad]).reshape(EC, CHUNK)

    s1 = prep(edge_index_1[0])
    dd1 = prep(edge_index_1[1])
    s2 = prep(edge_index_2[0])
    dd2 = prep(edge_index_2[1])
    bt1 = batch_1.astype(jnp.int32).reshape(1, N)
    bt2 = batch_2.astype(jnp.int32).reshape(1, N)

    bn_scale = 1.0 / jnp.sqrt(jnp.float32(1.0 + 1e-5))
    fpad = jnp.zeros((NP - N, 128), jnp.float32)
    c1 = jnp.concatenate([features_1, fpad])
    c2 = jnp.concatenate([features_2, fpad])
    outs = []
    dims = [128, 64, 64]
    filters = [64, 64, 128]
    for i in range(3):
        lp = p["gnn"][i]
        mi = p["mlp_inner"][i]
        mo = p["mlp_outer"][i]
        a1, a2 = _agg_pair(c1, c2, s1, dd1, s2, dd2)
        c1, c2, o1, o2 = _make_layer(dims[i], filters[i])(
            c1, a1, c2, a2, bt1, bt2,
            lp["eps"].reshape(1, 1),
            lp["lin1"]["W"], lp["lin1"]["b"].reshape(1, -1),
            lp["lin2"]["W"], lp["lin2"]["b"].reshape(1, -1),
            (lp["bn_gamma"] * bn_scale).reshape(1, -1),
            lp["bn_beta"].reshape(1, -1),
            mi["W"], mi["b"].reshape(1, -1),
            mo["W"], mo["b"].reshape(1, -1),
        )
        outs.append((o1, o2))

    FL, TN = 128, 16
    ntnw = p["ntn_W"].transpose(2, 0, 1).reshape(TN * FL, FL)
    res = _make_head()(
        outs[0][0], outs[0][1], outs[1][0], outs[1][1], outs[2][0], outs[2][1],
        p["conv1"]["W"], p["conv1"]["b"].reshape(1, -1),
        p["conv2"]["W"], p["conv2"]["b"].reshape(1, -1),
        ntnw,
        p["ntn_Wb"].T,
        p["ntn_bias"].reshape(1, TN),
        p["score1"]["W"], p["score1"]["b"].reshape(1, -1),
        p["score2"]["W"], p["score2"]["b"].reshape(1, -1),
        p["sim1"]["W"], p["sim1"]["b"].reshape(1, -1),
        p["sim2"]["W"], p["sim2"]["b"].reshape(1, -1),
        p["alpha"].reshape(1, 1), p["beta"].reshape(1, 1),
    )
    return res.reshape(-1)
